# baseline (speedup 1.0000x reference)
import functools

import jax
import jax.numpy as jnp
from jax import lax
from jax.experimental import pallas as pl
from jax.experimental.pallas import tpu as pltpu
from jax.experimental.pallas import tpu_sc as plsc

N_NODES = 10000
N_EDGES = 320000
D_FEAT = 128
D_EDGE = 16
EMB = 128
LAYERS = 3
NUM_GRAPHS = 128
BN_EPS = 1e-5

NC = 2
NS = 16
NW = NC * NS
EDGES_PER_WORKER = N_EDGES // NW
CHUNK = 80
N_CHUNKS = EDGES_PER_WORKER // CHUNK
ROW_STRIDE = 624
ROW_COPY = 640


def _sc_edge_body(h_hbm, e_hbm, src_hbm, dst_hbm, zero_hbm, out_hbm,
                  srcv, dstv, ebuf, hbuf, aggr_sh,
                  srcsem, dsem, esem, gsem, ssem):
    c = lax.axis_index("c")
    s = lax.axis_index("s")
    wid = c * NS + s
    base = wid * EDGES_PER_WORKER
    row0 = s * ROW_STRIDE
    last = N_CHUNKS - 1

    def issue(k, b):
        off = base + k * CHUNK
        pltpu.async_copy(dst_hbm.at[pl.ds(off, CHUNK)], dstv.at[b],
                         dsem.at[b])
        pltpu.async_copy(e_hbm.at[pl.ds(off, CHUNK)], ebuf.at[b], esem.at[b])
        pltpu.async_copy(h_hbm.at[srcv.at[b]], hbuf.at[b], gsem.at[b])

    pltpu.sync_copy(src_hbm.at[pl.ds(base, CHUNK)], srcv.at[0])
    issue(0, 0)
    pltpu.async_copy(src_hbm.at[pl.ds(base + CHUNK, CHUNK)], srcv.at[1],
                     srcsem.at[1])
    pltpu.sync_copy(zero_hbm.at[pl.ds(row0, ROW_COPY)],
                    aggr_sh.at[pl.ds(row0, ROW_COPY)])
    plsc.subcore_barrier()

    def pair_body(i, carry):
        for b in range(2):
            k = i * 2 + b
            nb = 1 - b

            def sub_body():
                @pl.when(k >= 1)
                def _():
                    pltpu.make_async_copy(hbuf.at[nb],
                                          aggr_sh.at[pl.ds(0, CHUNK)],
                                          ssem.at[nb]).wait()

                @pl.when(k + 1 <= last)
                def _():
                    pltpu.make_async_copy(
                        src_hbm.at[pl.ds(base, CHUNK)], srcv.at[nb],
                        srcsem.at[nb]).wait()
                    issue(k + 1, nb)

                pltpu.make_async_copy(e_hbm.at[pl.ds(base, CHUNK)],
                                      ebuf.at[b], esem.at[b]).wait()
                pltpu.make_async_copy(e_hbm.at[pl.ds(base, CHUNK)],
                                      hbuf.at[b], gsem.at[b]).wait()

                @pl.when(k + 2 <= last)
                def _():
                    pltpu.async_copy(
                        src_hbm.at[pl.ds(base + (k + 2) * CHUNK, CHUNK)],
                        srcv.at[b], srcsem.at[b])

                @plsc.parallel_loop(0, CHUNK, unroll=2)
                def _(row):
                    for j in range(EMB // 16):
                        cs = pl.ds(16 * j, 16)
                        v = hbuf[b, row, cs] + ebuf[b, row, cs]
                        hbuf[b, row, cs] = jnp.maximum(v, 0.0)
                pltpu.make_async_copy(dst_hbm.at[pl.ds(base, CHUNK)],
                                      dstv.at[b], dsem.at[b]).wait()

                pltpu.async_copy(hbuf.at[b], aggr_sh.at[dstv.at[b]],
                                 ssem.at[b], add=True)

            if b == 0:
                sub_body()
            else:
                pl.when(k <= last)(sub_body)
        return carry

    lax.fori_loop(0, (N_CHUNKS + 1) // 2, pair_body, 0)
    pltpu.make_async_copy(hbuf.at[0], aggr_sh.at[pl.ds(0, CHUNK)],
                          ssem.at[0]).wait()
    plsc.subcore_barrier()
    pltpu.sync_copy(aggr_sh.at[pl.ds(row0, ROW_COPY)],
                    out_hbm.at[c, pl.ds(row0, ROW_COPY)])


@functools.lru_cache(maxsize=1)
def _sc_edge():
    return pl.kernel(
        _sc_edge_body,
        out_type=jax.ShapeDtypeStruct((NC, N_NODES, EMB), jnp.float32),
        mesh=plsc.VectorSubcoreMesh(core_axis_name="c", subcore_axis_name="s",
                                    num_cores=NC, num_subcores=NS),
        scratch_types=[
            pltpu.VMEM((2, CHUNK), jnp.int32),
            pltpu.VMEM((2, CHUNK), jnp.int32),
            pltpu.VMEM((2, CHUNK, EMB), jnp.float32),
            pltpu.VMEM((2, CHUNK, EMB), jnp.float32),
            pltpu.VMEM_SHARED((N_NODES, EMB), jnp.float32),
            pltpu.SemaphoreType.DMA((2,)),
            pltpu.SemaphoreType.DMA((2,)),
            pltpu.SemaphoreType.DMA((2,)),
            pltpu.SemaphoreType.DMA((2,)),
            pltpu.SemaphoreType.DMA((2,)),
        ],
    )


def _embed_body(x_ref, w_ref, b_ref, o_ref):
    o_ref[...] = jnp.dot(x_ref[...], w_ref[...],
                         preferred_element_type=jnp.float32) + b_ref[...]


def _embed(x, w, b):
    return pl.pallas_call(
        _embed_body,
        out_shape=jax.ShapeDtypeStruct((x.shape[0], EMB), jnp.float32),
    )(x, w, b.reshape(1, -1))


EDGE_BLK = 3200


def _edge_mlp_body(a_ref, w_ref, b_ref, o_ref):
    o_ref[...] = jnp.dot(a_ref[...], w_ref[...],
                         preferred_element_type=jnp.float32) + b_ref[...]


def _edge_mlp_one(edge_attr, w, b):
    nblk = N_EDGES // EDGE_BLK
    return pl.pallas_call(
        _edge_mlp_body,
        grid=(nblk,),
        in_specs=[
            pl.BlockSpec((EDGE_BLK, D_EDGE), lambda i: (i, 0)),
            pl.BlockSpec((D_EDGE, EMB), lambda i: (0, 0)),
            pl.BlockSpec((1, EMB), lambda i: (0, 0)),
        ],
        out_specs=pl.BlockSpec((EDGE_BLK, EMB), lambda i: (i, 0)),
        out_shape=jax.ShapeDtypeStruct((N_EDGES, EMB), jnp.float32),
    )(edge_attr, w, b.reshape(1, EMB))


def _bn_in_kernel(u, g, b):
    m = jnp.mean(u, axis=0, keepdims=True)
    v = jnp.mean((u - m) * (u - m), axis=0, keepdims=True)
    return g * (u - m) * lax.rsqrt(v + BN_EPS) + b


def _node_mlp_body(relu_last, h_ref, aggr_ref, sc_ref, w1_ref, b1_ref,
                   g1_ref, be1_ref, w2_ref, b2_ref, bg_ref, bb_ref, o_ref):
    z = sc_ref[0, 0] * h_ref[...] + aggr_ref[0] + aggr_ref[1]
    u = jnp.dot(z, w1_ref[...], preferred_element_type=jnp.float32) + b1_ref[...]
    u = jnp.maximum(_bn_in_kernel(u, g1_ref[...], be1_ref[...]), 0.0)
    w = jnp.dot(u, w2_ref[...], preferred_element_type=jnp.float32) + b2_ref[...]
    w = _bn_in_kernel(w, bg_ref[...], bb_ref[...])
    if relu_last:
        w = jnp.maximum(w, 0.0)
    o_ref[...] = w


def _node_mlp(h, aggr, scale, w1, b1, g1, be1, w2, b2, bg, bb, relu_last):
    return pl.pallas_call(
        functools.partial(_node_mlp_body, relu_last),
        out_shape=jax.ShapeDtypeStruct((N_NODES, EMB), jnp.float32),
    )(h, aggr, scale, w1, b1.reshape(1, -1), g1.reshape(1, -1),
      be1.reshape(1, -1), w2, b2.reshape(1, -1), bg.reshape(1, -1),
      bb.reshape(1, -1))


def _separator_body(h_ref, batch_ref, sw1_ref, sb1_ref, sg_ref, sbe_ref,
                    sw2_ref, sb2_ref, score_ref, pos_ref, neg_ref):
    s = jnp.dot(h_ref[...], sw1_ref[...],
                preferred_element_type=jnp.float32) + sb1_ref[...]
    s = jnp.maximum(_bn_in_kernel(s, sg_ref[...], sbe_ref[...]), 0.0)
    t = jnp.dot(s, sw2_ref[...], preferred_element_type=jnp.float32) + sb2_ref[...]
    score = jax.nn.sigmoid(t)
    score_ref[...] = score
    pos_node = jnp.mean(score, axis=1, keepdims=True)
    gids = lax.broadcasted_iota(jnp.int32, (1, NUM_GRAPHS), 1)
    onehot = (batch_ref[...] == gids).astype(jnp.float32)
    pos_b = jnp.sum(onehot * pos_node, axis=0, keepdims=True)
    cnt = jnp.sum(onehot, axis=0, keepdims=True)
    pos_ref[...] = pos_b + 1e-8
    neg_ref[...] = (cnt - pos_b) + 1e-8


def _separator(h, batch2d, p):
    return pl.pallas_call(
        _separator_body,
        out_shape=[
            jax.ShapeDtypeStruct((N_NODES, EMB), jnp.float32),
            jax.ShapeDtypeStruct((1, NUM_GRAPHS), jnp.float32),
            jax.ShapeDtypeStruct((1, NUM_GRAPHS), jnp.float32),
        ],
    )(h, batch2d, p['sw1'], p['sb1'].reshape(1, -1), p['sg'].reshape(1, -1),
      p['sbe'].reshape(1, -1), p['sw2'], p['sb2'].reshape(1, -1))


def kernel(x, edge_index, edge_attr, batch, params):
    p = params
    src = edge_index[0].astype(jnp.int32)
    dst = edge_index[1].astype(jnp.int32)
    batch2d = batch.astype(jnp.int32).reshape(N_NODES, 1)
    zeros = jnp.zeros((N_NODES, EMB), jnp.float32)

    h = _embed(x, p['atom_w'], p['atom_b'])
    e_layers = [_edge_mlp_one(edge_attr, p['bond_w'][l], p['bond_b'][l])
                for l in range(LAYERS)]

    for l in range(LAYERS):
        aggr = _sc_edge()(h, e_layers[l], src, dst, zeros)
        h = _node_mlp(
            h, aggr, (1.0 + p['eps'][l]).reshape(1, 1),
            p['w1'][l], p['b1'][l], p['g1'][l], p['be1'][l],
            p['w2'][l], p['b2'][l], p['bn_g'][l], p['bn_b'][l],
            relu_last=(l < LAYERS - 1))

    score, pos_b, neg_b = _separator(h, batch2d, p)
    return score, pos_b.reshape(NUM_GRAPHS), neg_b.reshape(NUM_GRAPHS)


# speedup vs baseline: 1.0093x; 1.0093x over previous
import functools

import jax
import jax.numpy as jnp
from jax import lax
from jax.experimental import pallas as pl
from jax.experimental.pallas import tpu as pltpu
from jax.experimental.pallas import tpu_sc as plsc

N_NODES = 10000
N_EDGES = 320000
D_FEAT = 128
D_EDGE = 16
EMB = 128
LAYERS = 3
NUM_GRAPHS = 128
BN_EPS = 1e-5

NC = 2
NS = 16
NW = NC * NS
EDGES_PER_WORKER = N_EDGES // NW
CHUNK = 80
N_CHUNKS = EDGES_PER_WORKER // CHUNK
ROW_STRIDE = 624
ROW_COPY = 640


def _sc_edge_body(h_hbm, e_hbm, src_hbm, dst_hbm, zero_hbm, out_hbm,
                  srcv, dstv, ebuf, hbuf, aggr_sh,
                  srcsem, dsem, esem, gsem, ssem):
    c = lax.axis_index("c")
    s = lax.axis_index("s")
    wid = c * NS + s
    base = wid * EDGES_PER_WORKER
    row0 = s * ROW_STRIDE
    last = N_CHUNKS - 1

    def issue(k, b):
        off = base + k * CHUNK
        pltpu.async_copy(dst_hbm.at[pl.ds(off, CHUNK)], dstv.at[b],
                         dsem.at[b])
        pltpu.async_copy(e_hbm.at[pl.ds(off, CHUNK)], ebuf.at[b], esem.at[b])
        pltpu.async_copy(h_hbm.at[srcv.at[b]], hbuf.at[b], gsem.at[b])

    pltpu.sync_copy(src_hbm.at[pl.ds(base, CHUNK)], srcv.at[0])
    issue(0, 0)
    pltpu.async_copy(src_hbm.at[pl.ds(base + CHUNK, CHUNK)], srcv.at[1],
                     srcsem.at[1])
    pltpu.sync_copy(zero_hbm.at[pl.ds(row0, ROW_COPY)],
                    aggr_sh.at[pl.ds(row0, ROW_COPY)])
    plsc.subcore_barrier()

    def pair_body(i, carry):
        for b in range(2):
            k = i * 2 + b
            nb = 1 - b

            def sub_body():
                @pl.when(k >= 1)
                def _():
                    pltpu.make_async_copy(hbuf.at[nb],
                                          aggr_sh.at[pl.ds(0, CHUNK)],
                                          ssem.at[nb]).wait()

                @pl.when(k + 1 <= last)
                def _():
                    pltpu.make_async_copy(
                        src_hbm.at[pl.ds(base, CHUNK)], srcv.at[nb],
                        srcsem.at[nb]).wait()
                    issue(k + 1, nb)

                pltpu.make_async_copy(e_hbm.at[pl.ds(base, CHUNK)],
                                      ebuf.at[b], esem.at[b]).wait()
                pltpu.make_async_copy(e_hbm.at[pl.ds(base, CHUNK)],
                                      hbuf.at[b], gsem.at[b]).wait()

                @pl.when(k + 2 <= last)
                def _():
                    pltpu.async_copy(
                        src_hbm.at[pl.ds(base + (k + 2) * CHUNK, CHUNK)],
                        srcv.at[b], srcsem.at[b])

                def row_body(r, carry2):
                    for rr in range(2):
                        row = r * 2 + rr
                        for j in range(EMB // 16):
                            cs = pl.ds(16 * j, 16)
                            v = hbuf[b, row, cs] + ebuf[b, row, cs]
                            hbuf[b, row, cs] = jnp.maximum(v, 0.0)
                    return carry2

                lax.fori_loop(0, CHUNK // 2, row_body, 0)
                pltpu.make_async_copy(dst_hbm.at[pl.ds(base, CHUNK)],
                                      dstv.at[b], dsem.at[b]).wait()

                pltpu.async_copy(hbuf.at[b], aggr_sh.at[dstv.at[b]],
                                 ssem.at[b], add=True)

            if b == 0:
                sub_body()
            else:
                pl.when(k <= last)(sub_body)
        return carry

    lax.fori_loop(0, (N_CHUNKS + 1) // 2, pair_body, 0)
    pltpu.make_async_copy(hbuf.at[0], aggr_sh.at[pl.ds(0, CHUNK)],
                          ssem.at[0]).wait()
    plsc.subcore_barrier()
    pltpu.sync_copy(aggr_sh.at[pl.ds(row0, ROW_COPY)],
                    out_hbm.at[c, pl.ds(row0, ROW_COPY)])


@functools.lru_cache(maxsize=1)
def _sc_edge():
    return pl.kernel(
        _sc_edge_body,
        out_type=jax.ShapeDtypeStruct((NC, N_NODES, EMB), jnp.float32),
        mesh=plsc.VectorSubcoreMesh(core_axis_name="c", subcore_axis_name="s",
                                    num_cores=NC, num_subcores=NS),
        scratch_types=[
            pltpu.VMEM((2, CHUNK), jnp.int32),
            pltpu.VMEM((2, CHUNK), jnp.int32),
            pltpu.VMEM((2, CHUNK, EMB), jnp.float32),
            pltpu.VMEM((2, CHUNK, EMB), jnp.float32),
            pltpu.VMEM_SHARED((N_NODES, EMB), jnp.float32),
            pltpu.SemaphoreType.DMA((2,)),
            pltpu.SemaphoreType.DMA((2,)),
            pltpu.SemaphoreType.DMA((2,)),
            pltpu.SemaphoreType.DMA((2,)),
            pltpu.SemaphoreType.DMA((2,)),
        ],
    )


def _embed_body(x_ref, w_ref, b_ref, o_ref):
    o_ref[...] = jnp.dot(x_ref[...], w_ref[...],
                         preferred_element_type=jnp.float32) + b_ref[...]


def _embed(x, w, b):
    return pl.pallas_call(
        _embed_body,
        out_shape=jax.ShapeDtypeStruct((x.shape[0], EMB), jnp.float32),
    )(x, w, b.reshape(1, -1))


EDGE_BLK = 3200


def _edge_mlp_body(a_ref, w_ref, b_ref, o_ref):
    o_ref[...] = jnp.dot(a_ref[...], w_ref[...],
                         preferred_element_type=jnp.float32) + b_ref[...]


def _edge_mlp_one(edge_attr, w, b):
    nblk = N_EDGES // EDGE_BLK
    return pl.pallas_call(
        _edge_mlp_body,
        grid=(nblk,),
        in_specs=[
            pl.BlockSpec((EDGE_BLK, D_EDGE), lambda i: (i, 0)),
            pl.BlockSpec((D_EDGE, EMB), lambda i: (0, 0)),
            pl.BlockSpec((1, EMB), lambda i: (0, 0)),
        ],
        out_specs=pl.BlockSpec((EDGE_BLK, EMB), lambda i: (i, 0)),
        out_shape=jax.ShapeDtypeStruct((N_EDGES, EMB), jnp.float32),
    )(edge_attr, w, b.reshape(1, EMB))


def _bn_in_kernel(u, g, b):
    m = jnp.mean(u, axis=0, keepdims=True)
    v = jnp.mean((u - m) * (u - m), axis=0, keepdims=True)
    return g * (u - m) * lax.rsqrt(v + BN_EPS) + b


def _node_mlp_body(relu_last, h_ref, aggr_ref, sc_ref, w1_ref, b1_ref,
                   g1_ref, be1_ref, w2_ref, b2_ref, bg_ref, bb_ref, o_ref):
    z = sc_ref[0, 0] * h_ref[...] + aggr_ref[0] + aggr_ref[1]
    u = jnp.dot(z, w1_ref[...], preferred_element_type=jnp.float32) + b1_ref[...]
    u = jnp.maximum(_bn_in_kernel(u, g1_ref[...], be1_ref[...]), 0.0)
    w = jnp.dot(u, w2_ref[...], preferred_element_type=jnp.float32) + b2_ref[...]
    w = _bn_in_kernel(w, bg_ref[...], bb_ref[...])
    if relu_last:
        w = jnp.maximum(w, 0.0)
    o_ref[...] = w


def _node_mlp(h, aggr, scale, w1, b1, g1, be1, w2, b2, bg, bb, relu_last):
    return pl.pallas_call(
        functools.partial(_node_mlp_body, relu_last),
        out_shape=jax.ShapeDtypeStruct((N_NODES, EMB), jnp.float32),
    )(h, aggr, scale, w1, b1.reshape(1, -1), g1.reshape(1, -1),
      be1.reshape(1, -1), w2, b2.reshape(1, -1), bg.reshape(1, -1),
      bb.reshape(1, -1))


def _separator_body(h_ref, batch_ref, sw1_ref, sb1_ref, sg_ref, sbe_ref,
                    sw2_ref, sb2_ref, score_ref, pos_ref, neg_ref):
    s = jnp.dot(h_ref[...], sw1_ref[...],
                preferred_element_type=jnp.float32) + sb1_ref[...]
    s = jnp.maximum(_bn_in_kernel(s, sg_ref[...], sbe_ref[...]), 0.0)
    t = jnp.dot(s, sw2_ref[...], preferred_element_type=jnp.float32) + sb2_ref[...]
    score = jax.nn.sigmoid(t)
    score_ref[...] = score
    pos_node = jnp.mean(score, axis=1, keepdims=True)
    gids = lax.broadcasted_iota(jnp.int32, (1, NUM_GRAPHS), 1)
    onehot = (batch_ref[...] == gids).astype(jnp.float32)
    pos_b = jnp.sum(onehot * pos_node, axis=0, keepdims=True)
    cnt = jnp.sum(onehot, axis=0, keepdims=True)
    pos_ref[...] = pos_b + 1e-8
    neg_ref[...] = (cnt - pos_b) + 1e-8


def _separator(h, batch2d, p):
    return pl.pallas_call(
        _separator_body,
        out_shape=[
            jax.ShapeDtypeStruct((N_NODES, EMB), jnp.float32),
            jax.ShapeDtypeStruct((1, NUM_GRAPHS), jnp.float32),
            jax.ShapeDtypeStruct((1, NUM_GRAPHS), jnp.float32),
        ],
    )(h, batch2d, p['sw1'], p['sb1'].reshape(1, -1), p['sg'].reshape(1, -1),
      p['sbe'].reshape(1, -1), p['sw2'], p['sb2'].reshape(1, -1))


def kernel(x, edge_index, edge_attr, batch, params):
    p = params
    src = edge_index[0].astype(jnp.int32)
    dst = edge_index[1].astype(jnp.int32)
    batch2d = batch.astype(jnp.int32).reshape(N_NODES, 1)
    zeros = jnp.zeros((N_NODES, EMB), jnp.float32)

    h = _embed(x, p['atom_w'], p['atom_b'])
    e_layers = [_edge_mlp_one(edge_attr, p['bond_w'][l], p['bond_b'][l])
                for l in range(LAYERS)]

    for l in range(LAYERS):
        aggr = _sc_edge()(h, e_layers[l], src, dst, zeros)
        h = _node_mlp(
            h, aggr, (1.0 + p['eps'][l]).reshape(1, 1),
            p['w1'][l], p['b1'][l], p['g1'][l], p['be1'][l],
            p['w2'][l], p['b2'][l], p['bn_g'][l], p['bn_b'][l],
            relu_last=(l < LAYERS - 1))

    score, pos_b, neg_b = _separator(h, batch2d, p)
    return score, pos_b.reshape(NUM_GRAPHS), neg_b.reshape(NUM_GRAPHS)
